# baseline (speedup 1.0000x reference)
import functools

import jax
import jax.numpy as jnp
from jax import lax
from jax.experimental import pallas as pl
from jax.experimental.pallas import tpu as pltpu
from jax.experimental.pallas import tpu_sc as plsc


def kernel(inputs):
    B, T, D = inputs.shape
    info = plsc.get_sparse_core_info()
    NC, NS = info.num_cores, info.num_subcores
    NW = NC * NS
    chunks_per_b = NW // B
    chunk = D // chunks_per_b

    mesh = plsc.VectorSubcoreMesh(core_axis_name="c", subcore_axis_name="s")

    @functools.partial(
        pl.kernel,
        mesh=mesh,
        out_type=jax.ShapeDtypeStruct((B, D), inputs.dtype),
    )
    def extract(in_hbm, out_hbm):
        wid = lax.axis_index("s") * NC + lax.axis_index("c")
        b = wid // chunks_per_b
        off = (wid % chunks_per_b) * chunk
        pltpu.sync_copy(
            in_hbm.at[b, 0, pl.ds(off, chunk)],
            out_hbm.at[b, pl.ds(off, chunk)],
        )

    return extract(inputs)


# speedup vs baseline: 1.1439x; 1.1439x over previous
import functools

import jax
import jax.numpy as jnp
from jax import lax
from jax.experimental import pallas as pl
from jax.experimental.pallas import tpu as pltpu
from jax.experimental.pallas import tpu_sc as plsc


def kernel(inputs):
    B, T, D = inputs.shape

    mesh = plsc.ScalarSubcoreMesh(axis_name="c", num_cores=1)

    @functools.partial(
        pl.kernel,
        mesh=mesh,
        out_type=jax.ShapeDtypeStruct((B, D), inputs.dtype),
    )
    def extract(in_hbm, out_hbm):
        pltpu.sync_copy(in_hbm.at[:, 0, :], out_hbm)

    return extract(inputs)


# speedup vs baseline: 9.5466x; 8.3456x over previous
import jax
import jax.numpy as jnp
from jax.experimental import pallas as pl
from jax.experimental.pallas import tpu as pltpu


def kernel(inputs):
    B, T, D = inputs.shape

    def body(in_ref, out_ref, sem):
        pltpu.make_async_copy(in_ref.at[:, 0, :], out_ref, sem).start()
        pltpu.make_async_copy(in_ref.at[:, 0, :], out_ref, sem).wait()

    return pl.pallas_call(
        body,
        out_shape=jax.ShapeDtypeStruct((B, D), inputs.dtype),
        in_specs=[pl.BlockSpec(memory_space=pl.ANY)],
        out_specs=pl.BlockSpec(memory_space=pl.ANY),
        scratch_shapes=[pltpu.SemaphoreType.DMA],
    )(inputs)


# speedup vs baseline: 9.5714x; 1.0026x over previous
import jax
import jax.numpy as jnp
from jax.experimental import pallas as pl
from jax.experimental.pallas import tpu as pltpu


def kernel(inputs):
    B, T, D = inputs.shape

    def body(in_ref, out_ref, sem):
        for b in range(B):
            pltpu.make_async_copy(in_ref.at[b, 0, :], out_ref.at[b], sem).start()
        for b in range(B):
            pltpu.make_async_copy(in_ref.at[b, 0, :], out_ref.at[b], sem).wait()

    return pl.pallas_call(
        body,
        out_shape=jax.ShapeDtypeStruct((B, D), inputs.dtype),
        in_specs=[pl.BlockSpec(memory_space=pl.ANY)],
        out_specs=pl.BlockSpec(memory_space=pl.ANY),
        scratch_shapes=[pltpu.SemaphoreType.DMA],
    )(inputs)
